# Initial kernel scaffold; baseline (speedup 1.0000x reference)
import jax
import jax.numpy as jnp
from jax.experimental import pallas as pl


def kernel(x, router_W, W_fc, W_proj):
    raise NotImplementedError("write your pallas kernel here")


# speedup vs baseline: 1.6723x; 1.6723x over previous
import functools

import jax
import jax.numpy as jnp
from jax.experimental import pallas as pl
from jax.experimental.pallas import tpu as pltpu

N_EXPERTS = 8
TOP_K = 2
N_EMBD = 768
HIDDEN = 2048
N_TOKENS = 2048
T_TILE = 256
N_TILES = N_TOKENS // T_TILE


def _router_body(x_ref, rw_ref, wall_ref, laux_ref, zloss_ref):
    x = x_ref[...]
    rw = rw_ref[...]
    logits = jnp.dot(x, rw, preferred_element_type=jnp.float32)
    m = jnp.max(logits, axis=-1, keepdims=True)
    ex = jnp.exp(logits - m)
    se = jnp.sum(ex, axis=-1, keepdims=True)
    probs = ex / se

    cols = jax.lax.broadcasted_iota(jnp.int32, probs.shape, 1)
    w1 = jnp.max(probs, axis=-1, keepdims=True)
    i1 = jnp.argmax(probs, axis=-1)
    is1 = cols == i1[:, None]
    probs2 = jnp.where(is1, -jnp.inf, probs)
    w2 = jnp.max(probs2, axis=-1, keepdims=True)
    i2 = jnp.argmax(probs2, axis=-1)
    is2 = cols == i2[:, None]
    denom = w1 + w2 + 1e-9
    wall = (jnp.where(is1, w1, 0.0) + jnp.where(is2, w2, 0.0)) / denom
    wall_ref[...] = wall

    n = jnp.float32(x.shape[0])
    load = jnp.sum(is1.astype(jnp.float32), axis=0) / n
    importance = jnp.mean(probs, axis=0)
    laux_ref[0, 0] = N_EXPERTS * jnp.sum(load * importance)
    lse = m[:, 0] + jnp.log(se[:, 0])
    zloss_ref[0, 0] = jnp.mean(lse * lse)


def _ffn_body(x_ref, wfc_ref, wproj_ref, wall_ref, y_ref, acc_ref):
    e = pl.program_id(0)
    t = pl.program_id(1)
    x = x_ref[...]
    h = jnp.dot(x, wfc_ref[0], preferred_element_type=jnp.float32)
    gate = h[:, :HIDDEN]
    val = h[:, HIDDEN:]
    act = gate * jax.nn.sigmoid(gate) * val
    y_e = jnp.dot(act, wproj_ref[0], preferred_element_type=jnp.float32)
    wall = wall_ref[...]
    cols = jax.lax.broadcasted_iota(jnp.int32, wall.shape, 1)
    w = jnp.sum(jnp.where(cols == e, wall, 0.0), axis=1, keepdims=True)
    part = y_e * w
    sl = pl.ds(t * T_TILE, T_TILE)

    @pl.when(e == 0)
    def _():
        acc_ref[sl, :] = part

    @pl.when(e > 0)
    def _():
        acc_ref[sl, :] = acc_ref[sl, :] + part

    @pl.when(e == N_EXPERTS - 1)
    def _():
        y_ref[...] = acc_ref[sl, :]


def kernel(x, router_W, W_fc, W_proj):
    B, T, C = x.shape
    N = B * T
    x_flat = x.reshape(N, C)

    wall, laux, zloss = pl.pallas_call(
        _router_body,
        out_shape=(
            jax.ShapeDtypeStruct((N, N_EXPERTS), jnp.float32),
            jax.ShapeDtypeStruct((1, 1), jnp.float32),
            jax.ShapeDtypeStruct((1, 1), jnp.float32),
        ),
        in_specs=[
            pl.BlockSpec(memory_space=pltpu.VMEM),
            pl.BlockSpec(memory_space=pltpu.VMEM),
        ],
        out_specs=(
            pl.BlockSpec(memory_space=pltpu.VMEM),
            pl.BlockSpec(memory_space=pltpu.SMEM),
            pl.BlockSpec(memory_space=pltpu.SMEM),
        ),
    )(x_flat, router_W)

    y_flat = pl.pallas_call(
        _ffn_body,
        grid=(N_EXPERTS, N_TILES),
        in_specs=[
            pl.BlockSpec((T_TILE, C), lambda e, t: (t, 0)),
            pl.BlockSpec((1, C, 2 * HIDDEN), lambda e, t: (e, 0, 0)),
            pl.BlockSpec((1, HIDDEN, C), lambda e, t: (e, 0, 0)),
            pl.BlockSpec((T_TILE, N_EXPERTS), lambda e, t: (t, 0)),
        ],
        out_specs=pl.BlockSpec((T_TILE, C), lambda e, t: (t, 0)),
        out_shape=jax.ShapeDtypeStruct((N, C), jnp.float32),
        scratch_shapes=[pltpu.VMEM((N, C), jnp.float32)],
    )(x_flat, W_fc, W_proj, wall)

    return (y_flat.reshape(B, T, C), laux[0, 0], zloss[0, 0])
